# Initial kernel scaffold; baseline (speedup 1.0000x reference)
import jax
import jax.numpy as jnp
from jax.experimental import pallas as pl


def kernel(v, v_mask, coord, adj_matrix, top_ind, W, mean_rho, mean_theta, precision_rho, precision_theta):
    raise NotImplementedError("write your pallas kernel here")


# speedup vs baseline: 23.2547x; 23.2547x over previous
import functools
import math

import jax
import jax.numpy as jnp
from jax.experimental import pallas as pl

_B, _N, _FEAT = 64, 36, 2048
_K = 16
_NK = 8
_MID = 1024
_BM = 16


def _graph_conv_body(rho_ref, theta_ref, adj_ref, ti_ref, v_ref, wt_ref,
                     mr_ref, mt_ref, pr_ref, pt_ref, out_ref):
    n, nk, k = _N, _NK, _K
    wv = jnp.dot(v_ref[...], wt_ref[...], preferred_element_type=jnp.float32)

    mr = mr_ref[...]
    mt = mt_ref[...]
    pr = pr_ref[...]
    pt = pt_ref[...]

    iota_j = jax.lax.broadcasted_iota(jnp.int32, (n, n), 1)

    for b in range(_BM):
        rho = rho_ref[b]
        theta = theta_ref[b]

        ws = []
        for i in range(nk):
            d = (rho - mr[0, i]) ** 2
            w_r = jnp.exp(-0.5 * d / (1e-14 + pr[0, i] ** 2))
            fa = jnp.abs(theta - mt[0, i])
            sa = jnp.abs(2.0 * math.pi - fa)
            ang = jnp.minimum(fa, sa)
            w_t = jnp.exp(-0.5 * ang * ang / (1e-14 + pt[0, i] ** 2))
            w = w_r * w_t
            w = jnp.where(jnp.isnan(w), 0.0, w)
            ws.append(w)
        wsum = ws[0]
        for i in range(1, nk):
            wsum = wsum + ws[i]
        inv = 1.0 / (wsum + 1e-14)

        acc = jnp.zeros((n, n), dtype=jnp.float32)
        for kk in range(k):
            idx = ti_ref[b, :, kk][:, None]
            val = adj_ref[b, :, kk][:, None]
            acc = acc + jnp.where(iota_j == idx, val, 0.0)
        scaled = acc * inv

        base = b * n
        wv_b = wv[base:base + n, :]
        for i in range(nk):
            mi = ws[i] * scaled
            out_ref[base:base + n, i * 128:(i + 1) * 128] = jnp.dot(
                mi, wv_b[:, i * 128:(i + 1) * 128],
                preferred_element_type=jnp.float32)


@jax.jit
def _graph_conv(v, rho, theta, adj, ti, wcat_t, mr, mt, pr, pt):
    grid = _B // _BM
    out = pl.pallas_call(
        _graph_conv_body,
        grid=(grid,),
        in_specs=[
            pl.BlockSpec((_BM, _N, _N), lambda i: (i, 0, 0)),
            pl.BlockSpec((_BM, _N, _N), lambda i: (i, 0, 0)),
            pl.BlockSpec((_BM, _N, _K), lambda i: (i, 0, 0)),
            pl.BlockSpec((_BM, _N, _K), lambda i: (i, 0, 0)),
            pl.BlockSpec((_BM * _N, _FEAT), lambda i: (i, 0)),
            pl.BlockSpec((_FEAT, _MID), lambda i: (0, 0)),
            pl.BlockSpec((1, _NK), lambda i: (0, 0)),
            pl.BlockSpec((1, _NK), lambda i: (0, 0)),
            pl.BlockSpec((1, _NK), lambda i: (0, 0)),
            pl.BlockSpec((1, _NK), lambda i: (0, 0)),
        ],
        out_specs=pl.BlockSpec((_BM * _N, _MID), lambda i: (i, 0)),
        out_shape=jax.ShapeDtypeStruct((_B * _N, _MID), jnp.float32),
    )(rho, theta, adj, ti, v, wcat_t, mr, mt, pr, pt)
    return out.reshape(_B, _N, _MID)


def kernel(v, v_mask, coord, adj_matrix, top_ind, W, mean_rho, mean_theta,
           precision_rho, precision_theta):
    del v_mask
    rho = coord[:, :, :, 0]
    theta = coord[:, :, :, 1]
    v2 = v.reshape(_B * _N, _FEAT)
    wcat_t = W.reshape(_MID, _FEAT).T
    ti = top_ind.astype(jnp.int32)
    mr = mean_rho.reshape(1, _NK)
    mt = mean_theta.reshape(1, _NK)
    pr = precision_rho.reshape(1, _NK)
    pt = precision_theta.reshape(1, _NK)
    return _graph_conv(v2, rho, theta, adj_matrix, ti, wcat_t, mr, mt, pr, pt)
